# baseline (speedup 1.0000x reference)
import functools

import jax
import jax.numpy as jnp
from jax import lax
from jax.experimental import pallas as pl
from jax.experimental.pallas import tpu as pltpu
from jax.experimental.pallas import tpu_sc as plsc

N = 10000
E = 320000
D = 128
H = 8
DH = 16
F = H * DH
FW = F + 2 * H

NC = 2
NS = 16
L = 16

NP = 10016
ROWS_PER_TILE = NP // NS
CH = 96
TCH = 105
EPAD = NC * NS * TCH * CH
EIHALF = EPAD + CH
DUMMY = N


def _prologue_body(x_ref, w_ref, al_ref, ar_ref, tbl_ref, rle_ref):
    feat = jnp.dot(x_ref[...], w_ref[...], preferred_element_type=jnp.float32)
    el = jnp.dot(feat, al_ref[...], preferred_element_type=jnp.float32)
    er = jnp.dot(feat, ar_ref[...], preferred_element_type=jnp.float32)
    tbl_ref[...] = jnp.concatenate([feat, el, el], axis=1)
    rle_ref[...] = jnp.concatenate([er, er], axis=1)


def _prologue(x_pad, W, Al, Ar):
    blk = 2504
    return pl.pallas_call(
        _prologue_body,
        grid=(NP // blk,),
        in_specs=[
            pl.BlockSpec((blk, D), lambda i: (i, 0)),
            pl.BlockSpec((D, F), lambda i: (0, 0)),
            pl.BlockSpec((F, H), lambda i: (0, 0)),
            pl.BlockSpec((F, H), lambda i: (0, 0)),
        ],
        out_specs=[
            pl.BlockSpec((blk, FW), lambda i: (i, 0)),
            pl.BlockSpec((blk, 2 * H), lambda i: (i, 0)),
        ],
        out_shape=[
            jax.ShapeDtypeStruct((NP, FW), jnp.float32),
            jax.ShapeDtypeStruct((NP, 2 * H), jnp.float32),
        ],
    )(x_pad, W, Al, Ar)


def _edge_body(tbl_hbm, rle_hbm, ei_hbm, acc_out,
               srcb, dstb, dsc, erv, fv, acc_sh,
               s_src, s_dst, s_er, s_fv):
    c = lax.axis_index("c")
    s = lax.axis_index("s")
    wid = c * NS + s
    row0 = s * ROWS_PER_TILE
    base = wid * TCH
    z16 = jnp.zeros((L,), jnp.float32)

    def off_src(t):
        return pl.multiple_of((base + t) * CH, CH)

    def off_dst(t):
        return pl.multiple_of(EIHALF + (base + t) * CH, CH)

    def _zero_fv(i, _):
        for j in range(FW // L):
            fv[0][i, pl.ds(j * L, L)] = z16
        return 0
    lax.fori_loop(0, CH, _zero_fv, 0)
    for k in range(ROWS_PER_TILE // CH):
        pltpu.sync_copy(fv[0], acc_sh.at[pl.ds(row0 + k * CH, CH)])
    _rem = ROWS_PER_TILE % CH
    if _rem:
        pltpu.sync_copy(
            fv[0].at[pl.ds(0, _rem)],
            acc_sh.at[pl.ds(row0 + (ROWS_PER_TILE // CH) * CH, _rem)])
    plsc.subcore_barrier()

    def issue_idx(t, b):
        pltpu.async_copy(ei_hbm.at[pl.ds(off_src(t), CH)], srcb[b], s_src[b])
        pltpu.async_copy(ei_hbm.at[pl.ds(off_dst(t), CH)], dstb[b], s_dst[b])

    def wait_idx(b):
        pltpu.make_async_copy(ei_hbm.at[pl.ds(0, CH)], srcb[b], s_src[b]).wait()
        pltpu.make_async_copy(ei_hbm.at[pl.ds(0, CH)], dstb[b], s_dst[b]).wait()

    def mask_idx(b):
        for k in range(CH // L):
            sl = pl.ds(k * L, L)
            sv = srcb[b][sl]
            dv = dstb[b][sl]
            m = sv == dv
            srcb[b][sl] = jnp.where(m, DUMMY, sv)
            dm = jnp.where(m, DUMMY, dv)
            dstb[b][sl] = dm
            dsc[b][sl] = dm

    def issue_gathers(b):
        pltpu.async_copy(tbl_hbm.at[srcb[b]], fv[b], s_fv[b])
        pltpu.async_copy(rle_hbm.at[dstb[b]], erv[b], s_er[b])

    def wait_gathers(b):
        pltpu.make_async_copy(tbl_hbm.at[srcb[b]], fv[b], s_fv[b]).wait()
        pltpu.make_async_copy(rle_hbm.at[dstb[b]], erv[b], s_er[b]).wait()

    def compute(b):
        def _edge(i, _):
            sv = fv[b][i, pl.ds(F, L)] + erv[b][i, :]
            ee = jnp.exp(jnp.maximum(sv, 0.2 * sv))
            fv[b][i, pl.ds(F, L)] = ee
            for j in range(F // L):
                sl = pl.ds(j * L, L)
                fv[b][i, sl] = fv[b][i, sl] * ee
            return 0
        lax.fori_loop(0, CH, _edge, 0, unroll=8)

    def step(t, b):
        wait_idx(b ^ 1)
        mask_idx(b ^ 1)
        issue_gathers(b ^ 1)
        wait_gathers(b)
        issue_idx(t + 2, b)
        compute(b)
        pltpu.sync_copy(fv[b], acc_sh.at[dsc[b]], add=True)

    issue_idx(0, 0)
    wait_idx(0)
    mask_idx(0)
    issue_gathers(0)
    issue_idx(1, 1)

    def _pair(g, _):
        step(2 * g, 0)
        step(2 * g + 1, 1)
        return 0
    lax.fori_loop(0, (TCH - 1) // 2, _pair, 0)

    wait_idx(1)
    wait_gathers(0)
    compute(0)
    pltpu.sync_copy(fv[0], acc_sh.at[dsc[0]], add=True)

    plsc.subcore_barrier()
    rows = pl.ds(row0, ROWS_PER_TILE)
    pltpu.sync_copy(acc_sh.at[rows], acc_out.at[c, rows])


def _edge_pass(tbl, rle, ei_flat):
    mesh = plsc.VectorSubcoreMesh(
        core_axis_name="c", subcore_axis_name="s", num_cores=NC, num_subcores=NS)
    kfn = pl.kernel(
        _edge_body,
        out_type=jax.ShapeDtypeStruct((NC, NP, FW), jnp.float32),
        mesh=mesh,
        scratch_types=[
            [pltpu.VMEM((CH,), jnp.int32) for _ in range(2)],
            [pltpu.VMEM((CH,), jnp.int32) for _ in range(2)],
            [pltpu.VMEM((CH,), jnp.int32) for _ in range(2)],
            [pltpu.VMEM((CH, 2 * H), jnp.float32) for _ in range(2)],
            [pltpu.VMEM((CH, FW), jnp.float32) for _ in range(2)],
            pltpu.VMEM_SHARED((NP, FW), jnp.float32),
            [pltpu.SemaphoreType.DMA for _ in range(2)],
            [pltpu.SemaphoreType.DMA for _ in range(2)],
            [pltpu.SemaphoreType.DMA for _ in range(2)],
            [pltpu.SemaphoreType.DMA for _ in range(2)],
        ],
        compiler_params=pltpu.CompilerParams(use_tc_tiling_on_sc=False),
    )
    return kfn(tbl, rle, ei_flat)


def _epilogue_body(acc0_ref, acc1_ref, tbl_ref, rle_ref, b8_ref, p_ref,
                   out_ref):
    a0 = acc0_ref[...]
    a1 = acc1_ref[...]
    tbl = tbl_ref[...]
    sv = tbl[:, F:F + H] + rle_ref[:, :H]
    ee = jnp.exp(jnp.maximum(sv, 0.2 * sv))
    dent = a0[:, F:F + H] + a1[:, F:F + H] + ee
    b8 = b8_ref[...]
    eeb = jnp.dot(ee, b8, preferred_element_type=jnp.float32)
    denb = jnp.dot(dent, b8, preferred_element_type=jnp.float32) + 1e-16
    acc = a0[:, :F] + a1[:, :F] + eeb * tbl[:, :F]
    out_ref[...] = jnp.dot(acc / denb, p_ref[...],
                           preferred_element_type=jnp.float32)


def kernel(x, edge_index, W, attn_l, attn_r):
    x_pad = jnp.concatenate(
        [x, jnp.zeros((NP - N, D), jnp.float32)], axis=0)
    r = jnp.arange(F)
    q = (r % H) * DH + r // H
    Al = jnp.zeros((F, H), jnp.float32).at[r, r // DH].set(attn_l.reshape(F))
    Ar = jnp.zeros((F, H), jnp.float32).at[r, r // DH].set(attn_r.reshape(F))
    Wp = W[:, q]
    Alp = Al[q]
    Arp = Ar[q]
    B8 = (r[None, :] % H == jnp.arange(H)[:, None]).astype(jnp.float32)
    P = jnp.zeros((F, F), jnp.float32).at[(r % DH) * H + r // DH, r].set(1.0)

    fill = DUMMY + jnp.arange(EIHALF - E, dtype=jnp.int32) % (NP - N)
    fill_s = DUMMY + (jnp.arange(EIHALF - E, dtype=jnp.int32) + 1) % (NP - N)
    ei_flat = jnp.concatenate(
        [edge_index[0], fill_s, edge_index[1], fill])

    tbl, rle = _prologue(x_pad, Wp, Alp, Arp)
    acc = _edge_pass(tbl, rle, ei_flat)

    blk = 400
    out = pl.pallas_call(
        _epilogue_body,
        grid=(N // blk,),
        in_specs=[
            pl.BlockSpec((blk, FW), lambda i: (i, 0)),
            pl.BlockSpec((blk, FW), lambda i: (i, 0)),
            pl.BlockSpec((blk, FW), lambda i: (i, 0)),
            pl.BlockSpec((blk, 2 * H), lambda i: (i, 0)),
            pl.BlockSpec((H, F), lambda i: (0, 0)),
            pl.BlockSpec((F, F), lambda i: (0, 0)),
        ],
        out_specs=pl.BlockSpec((blk, F), lambda i: (i, 0)),
        out_shape=jax.ShapeDtypeStruct((N, F), jnp.float32),
    )(acc[0], acc[1], tbl, rle, B8, P)
    return out


# speedup vs baseline: 1.2141x; 1.2141x over previous
import functools

import jax
import jax.numpy as jnp
from jax import lax
from jax.experimental import pallas as pl
from jax.experimental.pallas import tpu as pltpu
from jax.experimental.pallas import tpu_sc as plsc

N = 10000
E = 320000
D = 128
H = 8
DH = 16
F = H * DH
FW = F + 2 * H

NC = 2
NS = 16
L = 16

NP = 10016
ROWS_PER_TILE = NP // NS
CH = 80
TCH = 126
EPAD = NC * NS * TCH * CH
EIHALF = EPAD
DUMMY = N


def _prologue_body(x_ref, w_ref, al_ref, ar_ref, tbl_ref, rle_ref):
    feat = jnp.dot(x_ref[...], w_ref[...], preferred_element_type=jnp.float32)
    el = jnp.dot(feat, al_ref[...], preferred_element_type=jnp.float32)
    er = jnp.dot(feat, ar_ref[...], preferred_element_type=jnp.float32)
    tbl_ref[...] = jnp.concatenate([feat, el, el], axis=1)
    rle_ref[...] = jnp.concatenate([er, er], axis=1)


def _prologue(x_pad, W, Al, Ar):
    blk = 2504
    return pl.pallas_call(
        _prologue_body,
        grid=(NP // blk,),
        in_specs=[
            pl.BlockSpec((blk, D), lambda i: (i, 0)),
            pl.BlockSpec((D, F), lambda i: (0, 0)),
            pl.BlockSpec((F, H), lambda i: (0, 0)),
            pl.BlockSpec((F, H), lambda i: (0, 0)),
        ],
        out_specs=[
            pl.BlockSpec((blk, FW), lambda i: (i, 0)),
            pl.BlockSpec((blk, 2 * H), lambda i: (i, 0)),
        ],
        out_shape=[
            jax.ShapeDtypeStruct((NP, FW), jnp.float32),
            jax.ShapeDtypeStruct((NP, 2 * H), jnp.float32),
        ],
    )(x_pad, W, Al, Ar)


def _edge_body(tbl_hbm, rle_hbm, ei_hbm, acc_out,
               srcb, dstb, dsc, erv, fv, acc_sh,
               s_src, s_dst, s_er, s_fv, s_sc):
    c = lax.axis_index("c")
    s = lax.axis_index("s")
    wid = c * NS + s
    row0 = s * ROWS_PER_TILE
    base = wid * TCH
    z16 = jnp.zeros((L,), jnp.float32)

    def off_src(t):
        return pl.multiple_of((base + t) * CH, CH)

    def off_dst(t):
        return pl.multiple_of(EIHALF + (base + t) * CH, CH)

    def _zero_fv(i, _):
        for j in range(FW // L):
            fv[0][i, pl.ds(j * L, L)] = z16
        return 0
    lax.fori_loop(0, CH, _zero_fv, 0)
    for k in range(ROWS_PER_TILE // CH):
        pltpu.sync_copy(fv[0], acc_sh.at[pl.ds(row0 + k * CH, CH)])
    _rem = ROWS_PER_TILE % CH
    if _rem:
        pltpu.sync_copy(
            fv[0].at[pl.ds(0, _rem)],
            acc_sh.at[pl.ds(row0 + (ROWS_PER_TILE // CH) * CH, _rem)])
    plsc.subcore_barrier()

    def issue_idx(t, b):
        pltpu.async_copy(ei_hbm.at[pl.ds(off_src(t), CH)], srcb[b], s_src[b])
        pltpu.async_copy(ei_hbm.at[pl.ds(off_dst(t), CH)], dstb[b], s_dst[b])

    def wait_idx(b):
        pltpu.make_async_copy(ei_hbm.at[pl.ds(0, CH)], srcb[b], s_src[b]).wait()
        pltpu.make_async_copy(ei_hbm.at[pl.ds(0, CH)], dstb[b], s_dst[b]).wait()

    def mask_idx(b):
        for k in range(CH // L):
            sl = pl.ds(k * L, L)
            sv = srcb[b][sl]
            dv = dstb[b][sl]
            m = sv == dv
            srcb[b][sl] = jnp.where(m, DUMMY, sv)
            dm = jnp.where(m, DUMMY, dv)
            dstb[b][sl] = dm
            dsc[b][sl] = dm

    def issue_gathers(b):
        pltpu.async_copy(tbl_hbm.at[srcb[b]], fv[b], s_fv[b])
        pltpu.async_copy(rle_hbm.at[dstb[b]], erv[b], s_er[b])

    def wait_gathers(b):
        pltpu.make_async_copy(tbl_hbm.at[srcb[b]], fv[b], s_fv[b]).wait()
        pltpu.make_async_copy(rle_hbm.at[dstb[b]], erv[b], s_er[b]).wait()

    def compute(b):
        def _edge(i, _):
            sv = fv[b][i, pl.ds(F, L)] + erv[b][i, :]
            ee = jnp.exp(jnp.maximum(sv, 0.2 * sv))
            fv[b][i, pl.ds(F, L)] = ee
            for j in range(F // L):
                sl = pl.ds(j * L, L)
                fv[b][i, sl] = fv[b][i, sl] * ee
            return 0
        lax.fori_loop(0, CH, _edge, 0, unroll=4)

    def issue_scatter(b):
        pltpu.async_copy(fv[b], acc_sh.at[dsc[b]], s_sc[b], add=True)

    def wait_scatter(b):
        pltpu.make_async_copy(fv[b], acc_sh.at[dsc[b]], s_sc[b]).wait()

    def step(t, b, first=False, last=False, prefetch=True):
        nb = (b + 1) % 3
        pb = (b + 2) % 3
        if not last:
            wait_idx(nb)
            mask_idx(nb)
            issue_gathers(nb)
        wait_gathers(b)
        if prefetch:
            issue_idx(t + 2, pb)
        compute(b)
        if not first:
            wait_scatter(pb)
        issue_scatter(b)

    issue_idx(0, 0)
    wait_idx(0)
    mask_idx(0)
    issue_gathers(0)
    issue_idx(1, 1)

    step(0, 0, first=True)

    def _triple(g, _):
        t = 3 * g + 1
        step(t, 1)
        step(t + 1, 2)
        step(t + 2, 0)
        return 0
    lax.fori_loop(0, (TCH - 3) // 3, _triple, 0)

    step(TCH - 2, 1, prefetch=False)
    step(TCH - 1, 2, last=True, prefetch=False)
    wait_scatter(2)

    plsc.subcore_barrier()
    rows = pl.ds(row0, ROWS_PER_TILE)
    pltpu.sync_copy(acc_sh.at[rows], acc_out.at[c, rows])


def _edge_pass(tbl, rle, ei_flat):
    mesh = plsc.VectorSubcoreMesh(
        core_axis_name="c", subcore_axis_name="s", num_cores=NC, num_subcores=NS)
    kfn = pl.kernel(
        _edge_body,
        out_type=jax.ShapeDtypeStruct((NC, NP, FW), jnp.float32),
        mesh=mesh,
        scratch_types=[
            [pltpu.VMEM((CH,), jnp.int32) for _ in range(3)],
            [pltpu.VMEM((CH,), jnp.int32) for _ in range(3)],
            [pltpu.VMEM((CH,), jnp.int32) for _ in range(3)],
            [pltpu.VMEM((CH, 2 * H), jnp.float32) for _ in range(3)],
            [pltpu.VMEM((CH, FW), jnp.float32) for _ in range(3)],
            pltpu.VMEM_SHARED((NP, FW), jnp.float32),
            [pltpu.SemaphoreType.DMA for _ in range(3)],
            [pltpu.SemaphoreType.DMA for _ in range(3)],
            [pltpu.SemaphoreType.DMA for _ in range(3)],
            [pltpu.SemaphoreType.DMA for _ in range(3)],
            [pltpu.SemaphoreType.DMA for _ in range(3)],
        ],
        compiler_params=pltpu.CompilerParams(use_tc_tiling_on_sc=False),
    )
    return kfn(tbl, rle, ei_flat)


def _epilogue_body(acc0_ref, acc1_ref, tbl_ref, rle_ref, b8_ref, p_ref,
                   out_ref):
    a0 = acc0_ref[...]
    a1 = acc1_ref[...]
    tbl = tbl_ref[...]
    sv = tbl[:, F:F + H] + rle_ref[:, :H]
    ee = jnp.exp(jnp.maximum(sv, 0.2 * sv))
    dent = a0[:, F:F + H] + a1[:, F:F + H] + ee
    b8 = b8_ref[...]
    eeb = jnp.dot(ee, b8, preferred_element_type=jnp.float32)
    denb = jnp.dot(dent, b8, preferred_element_type=jnp.float32) + 1e-16
    acc = a0[:, :F] + a1[:, :F] + eeb * tbl[:, :F]
    out_ref[...] = jnp.dot(acc / denb, p_ref[...],
                           preferred_element_type=jnp.float32)


def kernel(x, edge_index, W, attn_l, attn_r):
    x_pad = jnp.concatenate(
        [x, jnp.zeros((NP - N, D), jnp.float32)], axis=0)
    r = jnp.arange(F)
    q = (r % H) * DH + r // H
    Al = jnp.zeros((F, H), jnp.float32).at[r, r // DH].set(attn_l.reshape(F))
    Ar = jnp.zeros((F, H), jnp.float32).at[r, r // DH].set(attn_r.reshape(F))
    Wp = W[:, q]
    Alp = Al[q]
    Arp = Ar[q]
    B8 = (r[None, :] % H == jnp.arange(H)[:, None]).astype(jnp.float32)
    P = jnp.zeros((F, F), jnp.float32).at[(r % DH) * H + r // DH, r].set(1.0)

    fill = DUMMY + jnp.arange(EIHALF - E, dtype=jnp.int32) % (NP - N)
    fill_s = DUMMY + (jnp.arange(EIHALF - E, dtype=jnp.int32) + 1) % (NP - N)
    ei_flat = jnp.concatenate(
        [edge_index[0], fill_s, edge_index[1], fill])

    tbl, rle = _prologue(x_pad, Wp, Alp, Arp)
    acc = _edge_pass(tbl, rle, ei_flat)

    blk = 400
    out = pl.pallas_call(
        _epilogue_body,
        grid=(N // blk,),
        in_specs=[
            pl.BlockSpec((blk, FW), lambda i: (i, 0)),
            pl.BlockSpec((blk, FW), lambda i: (i, 0)),
            pl.BlockSpec((blk, FW), lambda i: (i, 0)),
            pl.BlockSpec((blk, 2 * H), lambda i: (i, 0)),
            pl.BlockSpec((H, F), lambda i: (0, 0)),
            pl.BlockSpec((F, F), lambda i: (0, 0)),
        ],
        out_specs=pl.BlockSpec((blk, F), lambda i: (i, 0)),
        out_shape=jax.ShapeDtypeStruct((N, F), jnp.float32),
    )(acc[0], acc[1], tbl, rle, B8, P)
    return out


# speedup vs baseline: 1.3057x; 1.0754x over previous
import functools

import jax
import jax.numpy as jnp
from jax import lax
from jax.experimental import pallas as pl
from jax.experimental.pallas import tpu as pltpu
from jax.experimental.pallas import tpu_sc as plsc

N = 10000
E = 320000
D = 128
H = 8
DH = 16
F = H * DH
FW = F + 2 * H

NC = 2
NS = 16
L = 16

NP = 10016
ROWS_PER_TILE = NP // NS
CH = 80
TCH = 126
EPAD = NC * NS * TCH * CH
EIHALF = EPAD
DUMMY = N


def _prologue_body(x_ref, w_ref, al_ref, ar_ref, tbl_ref, rle_ref):
    feat = jnp.dot(x_ref[...], w_ref[...], preferred_element_type=jnp.float32)
    el = jnp.dot(feat, al_ref[...], preferred_element_type=jnp.float32)
    er = jnp.dot(feat, ar_ref[...], preferred_element_type=jnp.float32)
    tbl_ref[...] = jnp.concatenate([feat, el, el], axis=1)
    rle_ref[...] = jnp.concatenate([er, er], axis=1)


def _prologue(x_pad, W, Al, Ar):
    blk = 2504
    return pl.pallas_call(
        _prologue_body,
        grid=(NP // blk,),
        in_specs=[
            pl.BlockSpec((blk, D), lambda i: (i, 0)),
            pl.BlockSpec((D, F), lambda i: (0, 0)),
            pl.BlockSpec((F, H), lambda i: (0, 0)),
            pl.BlockSpec((F, H), lambda i: (0, 0)),
        ],
        out_specs=[
            pl.BlockSpec((blk, FW), lambda i: (i, 0)),
            pl.BlockSpec((blk, 2 * H), lambda i: (i, 0)),
        ],
        out_shape=[
            jax.ShapeDtypeStruct((NP, FW), jnp.float32),
            jax.ShapeDtypeStruct((NP, 2 * H), jnp.float32),
        ],
    )(x_pad, W, Al, Ar)


def _edge_body(tbl_hbm, rle_hbm, ei_hbm, acc_out,
               srcb, dstb, dsc, erv, fv, acc_sh,
               s_src, s_dst, s_er, s_fv, s_sc):
    c = lax.axis_index("c")
    s = lax.axis_index("s")
    wid = c * NS + s
    row0 = s * ROWS_PER_TILE
    base = wid * TCH
    z16 = jnp.zeros((L,), jnp.float32)

    def off_src(t):
        return pl.multiple_of((base + t) * CH, CH)

    def off_dst(t):
        return pl.multiple_of(EIHALF + (base + t) * CH, CH)

    def _zero_fv(i, _):
        for j in range(FW // L):
            fv[0][i, pl.ds(j * L, L)] = z16
        return 0
    lax.fori_loop(0, CH, _zero_fv, 0)
    for k in range(ROWS_PER_TILE // CH):
        pltpu.sync_copy(fv[0], acc_sh.at[pl.ds(row0 + k * CH, CH)])
    _rem = ROWS_PER_TILE % CH
    if _rem:
        pltpu.sync_copy(
            fv[0].at[pl.ds(0, _rem)],
            acc_sh.at[pl.ds(row0 + (ROWS_PER_TILE // CH) * CH, _rem)])
    plsc.subcore_barrier()

    def issue_idx(t, b):
        pltpu.async_copy(ei_hbm.at[pl.ds(off_src(t), CH)], srcb[b], s_src[b])
        pltpu.async_copy(ei_hbm.at[pl.ds(off_dst(t), CH)], dstb[b], s_dst[b])

    def wait_idx(b):
        pltpu.make_async_copy(ei_hbm.at[pl.ds(0, CH)], srcb[b], s_src[b]).wait()
        pltpu.make_async_copy(ei_hbm.at[pl.ds(0, CH)], dstb[b], s_dst[b]).wait()

    def mask_idx(b):
        for k in range(CH // L):
            sl = pl.ds(k * L, L)
            sv = srcb[b][sl]
            dv = dstb[b][sl]
            m = sv == dv
            srcb[b][sl] = jnp.where(m, DUMMY, sv)
            dm = jnp.where(m, DUMMY, dv)
            dstb[b][sl] = dm
            dsc[b][sl] = dm

    def issue_gathers(b):
        pltpu.async_copy(tbl_hbm.at[srcb[b]], fv[b], s_fv[b])
        pltpu.async_copy(rle_hbm.at[dstb[b]], erv[b], s_er[b])

    def wait_gathers(b):
        pltpu.make_async_copy(tbl_hbm.at[srcb[b]], fv[b], s_fv[b]).wait()
        pltpu.make_async_copy(rle_hbm.at[dstb[b]], erv[b], s_er[b]).wait()

    def compute(b):
        def _edge(i, _):
            sv = fv[b][i, pl.ds(F, L)] + erv[b][i, :]
            ee = jnp.exp(jnp.maximum(sv, 0.2 * sv))
            fv[b][i, pl.ds(F, L)] = ee
            for j in range(F // L):
                sl = pl.ds(j * L, L)
                fv[b][i, sl] = fv[b][i, sl] * ee
            return 0
        lax.fori_loop(0, CH, _edge, 0, unroll=4)

    def issue_scatter(b):
        pltpu.async_copy(fv[b], acc_sh.at[dsc[b]], s_sc[b], add=True)

    def wait_scatter(b):
        pltpu.make_async_copy(fv[b], acc_sh.at[dsc[b]], s_sc[b]).wait()

    def step(t, b, first=False, last=False, prefetch=True):
        nb = (b + 1) % 3
        pb = (b + 2) % 3
        if not last:
            wait_idx(nb)
            mask_idx(nb)
            issue_gathers(nb)
        wait_gathers(b)
        if prefetch:
            issue_idx(t + 2, pb)
        compute(b)
        if not first:
            wait_scatter(pb)
        issue_scatter(b)

    issue_idx(0, 0)
    wait_idx(0)
    mask_idx(0)
    issue_gathers(0)
    issue_idx(1, 1)

    step(0, 0, first=True)

    def _triple(g, _):
        t = 3 * g + 1
        step(t, 1)
        step(t + 1, 2)
        step(t + 2, 0)
        return 0
    lax.fori_loop(0, (TCH - 3) // 3, _triple, 0)

    step(TCH - 2, 1, prefetch=False)
    step(TCH - 1, 2, last=True, prefetch=False)
    wait_scatter(2)

    plsc.subcore_barrier()
    rows = pl.ds(row0, ROWS_PER_TILE)
    pltpu.sync_copy(acc_sh.at[rows], acc_out.at[c, rows])


def _edge_pass(tbl, rle, ei_flat):
    mesh = plsc.VectorSubcoreMesh(
        core_axis_name="c", subcore_axis_name="s", num_cores=NC, num_subcores=NS)
    kfn = pl.kernel(
        _edge_body,
        out_type=jax.ShapeDtypeStruct((NC, NP, FW), jnp.float32),
        mesh=mesh,
        scratch_types=[
            [pltpu.VMEM((CH,), jnp.int32) for _ in range(3)],
            [pltpu.VMEM((CH,), jnp.int32) for _ in range(3)],
            [pltpu.VMEM((CH,), jnp.int32) for _ in range(3)],
            [pltpu.VMEM((CH, 2 * H), jnp.float32) for _ in range(3)],
            [pltpu.VMEM((CH, FW), jnp.float32) for _ in range(3)],
            pltpu.VMEM_SHARED((NP, FW), jnp.float32),
            [pltpu.SemaphoreType.DMA for _ in range(3)],
            [pltpu.SemaphoreType.DMA for _ in range(3)],
            [pltpu.SemaphoreType.DMA for _ in range(3)],
            [pltpu.SemaphoreType.DMA for _ in range(3)],
            [pltpu.SemaphoreType.DMA for _ in range(3)],
        ],
        compiler_params=pltpu.CompilerParams(use_tc_tiling_on_sc=False),
    )
    return kfn(tbl, rle, ei_flat)


def _epilogue_body(acc0_ref, acc1_ref, tbl_ref, rle_ref, b8_ref, p_ref,
                   out_ref):
    a0 = acc0_ref[...]
    a1 = acc1_ref[...]
    tbl = tbl_ref[...]
    sv = tbl[:, F:F + H] + rle_ref[:, :H]
    ee = jnp.exp(jnp.maximum(sv, 0.2 * sv))
    dent = a0[:, F:F + H] + a1[:, F:F + H] + ee
    b8 = b8_ref[...]
    eeb = jnp.dot(ee, b8, preferred_element_type=jnp.float32)
    denb = jnp.dot(dent, b8, preferred_element_type=jnp.float32) + 1e-16
    acc = a0[:, :F] + a1[:, :F] + eeb * tbl[:, :F]
    out_ref[...] = jnp.dot(acc / denb, p_ref[...],
                           preferred_element_type=jnp.float32)


def kernel(x, edge_index, W, attn_l, attn_r):
    x_pad = jnp.concatenate(
        [x, jnp.zeros((NP - N, D), jnp.float32)], axis=0)
    r = jnp.arange(F)
    q = (r % H) * DH + r // H
    Wp = W[:, q]
    hsel = (r[:, None] % H == jnp.arange(H)[None, :]).astype(jnp.float32)
    Alp = attn_l.T.reshape(F)[:, None] * hsel
    Arp = attn_r.T.reshape(F)[:, None] * hsel
    B8 = hsel.T
    P = (r[:, None] == (r[None, :] % DH) * H + r[None, :] // DH
         ).astype(jnp.float32)

    fill = DUMMY + jnp.arange(EIHALF - E, dtype=jnp.int32) % (NP - N)
    fill_s = DUMMY + (jnp.arange(EIHALF - E, dtype=jnp.int32) + 1) % (NP - N)
    ei_flat = jnp.concatenate(
        [edge_index[0], fill_s, edge_index[1], fill])

    tbl, rle = _prologue(x_pad, Wp, Alp, Arp)
    acc = _edge_pass(tbl, rle, ei_flat)

    blk = 2000
    out = pl.pallas_call(
        _epilogue_body,
        grid=(N // blk,),
        in_specs=[
            pl.BlockSpec((blk, FW), lambda i: (i, 0)),
            pl.BlockSpec((blk, FW), lambda i: (i, 0)),
            pl.BlockSpec((blk, FW), lambda i: (i, 0)),
            pl.BlockSpec((blk, 2 * H), lambda i: (i, 0)),
            pl.BlockSpec((H, F), lambda i: (0, 0)),
            pl.BlockSpec((F, F), lambda i: (0, 0)),
        ],
        out_specs=pl.BlockSpec((blk, F), lambda i: (i, 0)),
        out_shape=jax.ShapeDtypeStruct((N, F), jnp.float32),
    )(acc[0], acc[1], tbl, rle, B8, P)
    return out


# speedup vs baseline: 1.3196x; 1.0106x over previous
import functools

import jax
import jax.numpy as jnp
from jax import lax
from jax.experimental import pallas as pl
from jax.experimental.pallas import tpu as pltpu
from jax.experimental.pallas import tpu_sc as plsc

N = 10000
E = 320000
D = 128
H = 8
DH = 16
F = H * DH
FW = F + 2 * H

NC = 2
NS = 16
L = 16

NP = 10016
ROWS_PER_TILE = NP // NS
CH = 80
TCH = 126
EPAD = NC * NS * TCH * CH
EIHALF = EPAD
DUMMY = N


def _prologue_body(x_ref, w_ref, al_ref, ar_ref, tbl_ref, rle_ref):
    feat = jnp.dot(x_ref[...], w_ref[...], preferred_element_type=jnp.float32)
    el = jnp.dot(feat, al_ref[...], preferred_element_type=jnp.float32)
    er = jnp.dot(feat, ar_ref[...], preferred_element_type=jnp.float32)
    tbl_ref[pl.ds(0, N), :] = jnp.concatenate([feat, el, el], axis=1)
    tbl_ref[pl.ds(N, NP - N), :] = jnp.zeros((NP - N, FW), jnp.float32)
    rle_ref[pl.ds(0, N), :] = jnp.concatenate([er, er], axis=1)
    rle_ref[pl.ds(N, NP - N), :] = jnp.zeros((NP - N, 2 * H), jnp.float32)


def _prologue(x, W, Al, Ar):
    return pl.pallas_call(
        _prologue_body,
        grid=(1,),
        in_specs=[
            pl.BlockSpec((N, D), lambda i: (0, 0)),
            pl.BlockSpec((D, F), lambda i: (0, 0)),
            pl.BlockSpec((F, H), lambda i: (0, 0)),
            pl.BlockSpec((F, H), lambda i: (0, 0)),
        ],
        out_specs=[
            pl.BlockSpec((NP, FW), lambda i: (0, 0)),
            pl.BlockSpec((NP, 2 * H), lambda i: (0, 0)),
        ],
        out_shape=[
            jax.ShapeDtypeStruct((NP, FW), jnp.float32),
            jax.ShapeDtypeStruct((NP, 2 * H), jnp.float32),
        ],
    )(x, W, Al, Ar)


def _edge_body(tbl_hbm, rle_hbm, ei_hbm, acc_out,
               srcb, dstb, dsc, erv, fv, acc_sh,
               s_src, s_dst, s_er, s_fv, s_sc):
    c = lax.axis_index("c")
    s = lax.axis_index("s")
    wid = c * NS + s
    row0 = s * ROWS_PER_TILE
    base = wid * TCH
    z16 = jnp.zeros((L,), jnp.float32)

    def off_src(t):
        return pl.multiple_of((base + t) * CH, CH)

    def off_dst(t):
        return pl.multiple_of(EIHALF + (base + t) * CH, CH)

    def _zero_fv(i, _):
        for j in range(FW // L):
            fv[0][i, pl.ds(j * L, L)] = z16
        return 0
    lax.fori_loop(0, CH, _zero_fv, 0)
    for k in range(ROWS_PER_TILE // CH):
        pltpu.sync_copy(fv[0], acc_sh.at[pl.ds(row0 + k * CH, CH)])
    _rem = ROWS_PER_TILE % CH
    if _rem:
        pltpu.sync_copy(
            fv[0].at[pl.ds(0, _rem)],
            acc_sh.at[pl.ds(row0 + (ROWS_PER_TILE // CH) * CH, _rem)])
    plsc.subcore_barrier()

    def issue_idx(t, b):
        pltpu.async_copy(ei_hbm.at[pl.ds(off_src(t), CH)], srcb[b], s_src[b])
        pltpu.async_copy(ei_hbm.at[pl.ds(off_dst(t), CH)], dstb[b], s_dst[b])

    def wait_idx(b):
        pltpu.make_async_copy(ei_hbm.at[pl.ds(0, CH)], srcb[b], s_src[b]).wait()
        pltpu.make_async_copy(ei_hbm.at[pl.ds(0, CH)], dstb[b], s_dst[b]).wait()

    def mask_idx(b):
        for k in range(CH // L):
            sl = pl.ds(k * L, L)
            sv = srcb[b][sl]
            dv = dstb[b][sl]
            m = sv == dv
            srcb[b][sl] = jnp.where(m, DUMMY, sv)
            dm = jnp.where(m, DUMMY, dv)
            dstb[b][sl] = dm
            dsc[b][sl] = dm

    def issue_gathers(b):
        pltpu.async_copy(tbl_hbm.at[srcb[b]], fv[b], s_fv[b])
        pltpu.async_copy(rle_hbm.at[dstb[b]], erv[b], s_er[b])

    def wait_gathers(b):
        pltpu.make_async_copy(tbl_hbm.at[srcb[b]], fv[b], s_fv[b]).wait()
        pltpu.make_async_copy(rle_hbm.at[dstb[b]], erv[b], s_er[b]).wait()

    def compute(b):
        def _edge(i, _):
            sv = fv[b][i, pl.ds(F, L)] + erv[b][i, :]
            ee = jnp.exp(jnp.maximum(sv, 0.2 * sv))
            fv[b][i, pl.ds(F, L)] = ee
            for j in range(F // L):
                sl = pl.ds(j * L, L)
                fv[b][i, sl] = fv[b][i, sl] * ee
            return 0
        lax.fori_loop(0, CH, _edge, 0, unroll=4)

    def issue_scatter(b):
        pltpu.async_copy(fv[b], acc_sh.at[dsc[b]], s_sc[b], add=True)

    def wait_scatter(b):
        pltpu.make_async_copy(fv[b], acc_sh.at[dsc[b]], s_sc[b]).wait()

    def step(t, b, first=False, last=False, prefetch=True):
        nb = (b + 1) % 3
        pb = (b + 2) % 3
        if not last:
            wait_idx(nb)
            mask_idx(nb)
            issue_gathers(nb)
        wait_gathers(b)
        if prefetch:
            issue_idx(t + 2, pb)
        compute(b)
        if not first:
            wait_scatter(pb)
        issue_scatter(b)

    issue_idx(0, 0)
    wait_idx(0)
    mask_idx(0)
    issue_gathers(0)
    issue_idx(1, 1)

    step(0, 0, first=True)

    def _triple(g, _):
        t = 3 * g + 1
        step(t, 1)
        step(t + 1, 2)
        step(t + 2, 0)
        return 0
    lax.fori_loop(0, (TCH - 3) // 3, _triple, 0)

    step(TCH - 2, 1, prefetch=False)
    step(TCH - 1, 2, last=True, prefetch=False)
    wait_scatter(2)

    plsc.subcore_barrier()
    rows = pl.ds(row0, ROWS_PER_TILE)
    pltpu.sync_copy(acc_sh.at[rows], acc_out.at[c, rows])


def _edge_pass(tbl, rle, ei_flat):
    mesh = plsc.VectorSubcoreMesh(
        core_axis_name="c", subcore_axis_name="s", num_cores=NC, num_subcores=NS)
    kfn = pl.kernel(
        _edge_body,
        out_type=jax.ShapeDtypeStruct((NC, NP, FW), jnp.float32),
        mesh=mesh,
        scratch_types=[
            [pltpu.VMEM((CH,), jnp.int32) for _ in range(3)],
            [pltpu.VMEM((CH,), jnp.int32) for _ in range(3)],
            [pltpu.VMEM((CH,), jnp.int32) for _ in range(3)],
            [pltpu.VMEM((CH, 2 * H), jnp.float32) for _ in range(3)],
            [pltpu.VMEM((CH, FW), jnp.float32) for _ in range(3)],
            pltpu.VMEM_SHARED((NP, FW), jnp.float32),
            [pltpu.SemaphoreType.DMA for _ in range(3)],
            [pltpu.SemaphoreType.DMA for _ in range(3)],
            [pltpu.SemaphoreType.DMA for _ in range(3)],
            [pltpu.SemaphoreType.DMA for _ in range(3)],
            [pltpu.SemaphoreType.DMA for _ in range(3)],
        ],
        compiler_params=pltpu.CompilerParams(use_tc_tiling_on_sc=False),
    )
    return kfn(tbl, rle, ei_flat)


def _epilogue_body(acc0_ref, acc1_ref, tbl_ref, rle_ref, b8_ref, p_ref,
                   out_ref):
    a0 = acc0_ref[...]
    a1 = acc1_ref[...]
    tbl = tbl_ref[...]
    sv = tbl[:, F:F + H] + rle_ref[:, :H]
    ee = jnp.exp(jnp.maximum(sv, 0.2 * sv))
    dent = a0[:, F:F + H] + a1[:, F:F + H] + ee
    b8 = b8_ref[...]
    eeb = jnp.dot(ee, b8, preferred_element_type=jnp.float32)
    denb = jnp.dot(dent, b8, preferred_element_type=jnp.float32) + 1e-16
    acc = a0[:, :F] + a1[:, :F] + eeb * tbl[:, :F]
    out_ref[...] = jnp.dot(acc / denb, p_ref[...],
                           preferred_element_type=jnp.float32)


def kernel(x, edge_index, W, attn_l, attn_r):
    r = jnp.arange(F)
    q = (r % H) * DH + r // H
    Wp = W[:, q]
    hsel = (r[:, None] % H == jnp.arange(H)[None, :]).astype(jnp.float32)
    Alp = attn_l.T.reshape(F)[:, None] * hsel
    Arp = attn_r.T.reshape(F)[:, None] * hsel
    B8 = hsel.T
    P = (r[:, None] == (r[None, :] % DH) * H + r[None, :] // DH
         ).astype(jnp.float32)

    fill = DUMMY + jnp.arange(EIHALF - E, dtype=jnp.int32) % (NP - N)
    fill_s = DUMMY + (jnp.arange(EIHALF - E, dtype=jnp.int32) + 1) % (NP - N)
    ei_flat = jnp.concatenate(
        [edge_index[0], fill_s, edge_index[1], fill])

    tbl, rle = _prologue(x, Wp, Alp, Arp)
    acc = _edge_pass(tbl, rle, ei_flat)

    blk = 2000
    out = pl.pallas_call(
        _epilogue_body,
        grid=(N // blk,),
        in_specs=[
            pl.BlockSpec((blk, FW), lambda i: (i, 0)),
            pl.BlockSpec((blk, FW), lambda i: (i, 0)),
            pl.BlockSpec((blk, FW), lambda i: (i, 0)),
            pl.BlockSpec((blk, 2 * H), lambda i: (i, 0)),
            pl.BlockSpec((H, F), lambda i: (0, 0)),
            pl.BlockSpec((F, F), lambda i: (0, 0)),
        ],
        out_specs=pl.BlockSpec((blk, F), lambda i: (i, 0)),
        out_shape=jax.ShapeDtypeStruct((N, F), jnp.float32),
    )(acc[0], acc[1], tbl, rle, B8, P)
    return out
